# baseline (speedup 1.0000x reference)
import math

import jax
import jax.numpy as jnp
from jax.experimental import pallas as pl
from jax.experimental.pallas import tpu as pltpu

VOCAB = 32000
N_TOK = 2048
PAD = 0
SMOOTHING = 0.1
CONF = 1.0 - SMOOTHING
EPS = SMOOTHING / (VOCAB - 2)
ROW_CONST = (VOCAB - 2) * EPS * math.log(EPS) + CONF * math.log(CONF)

RB = 64
CB = VOCAB


def _loss_kernel(tgt_ref, x_ref, out_ref):
    i = pl.program_id(0)

    @pl.when(i == 0)
    def _():
        out_ref[...] = jnp.zeros((1, 1), jnp.float32)

    x = x_ref[...]
    tgt = tgt_ref[...]
    valid = tgt != PAD

    rowsum = jnp.sum(x, axis=1, keepdims=True)
    cols = jax.lax.broadcasted_iota(jnp.int32, (RB, CB), 1)
    hit = cols == tgt
    xt = jnp.sum(jnp.where(hit, x, 0.0), axis=1, keepdims=True)

    contrib = ROW_CONST - EPS * rowsum + EPS * x[:, 0:1] - (CONF - EPS) * xt
    contrib = jnp.where(valid, contrib, 0.0)
    out_ref[...] += jnp.sum(contrib, axis=0, keepdims=True)


@jax.jit
def kernel(x, target):
    tgt = target.astype(jnp.int32).reshape(N_TOK, 1)
    out = pl.pallas_call(
        _loss_kernel,
        grid=(N_TOK // RB,),
        in_specs=[
            pl.BlockSpec((RB, 1), lambda i: (i, 0)),
            pl.BlockSpec((RB, CB), lambda i: (i, 0)),
        ],
        out_specs=pl.BlockSpec((1, 1), lambda i: (0, 0)),
        out_shape=jax.ShapeDtypeStruct((1, 1), jnp.float32),
        compiler_params=pltpu.CompilerParams(
            dimension_semantics=("arbitrary",),
        ),
    )(tgt, x)
    return out[0, 0]
